# speedup vs baseline: 22.9403x; 22.9403x over previous
import functools

import jax
import jax.numpy as jnp
from jax import lax
from jax.experimental import pallas as pl
from jax.experimental.pallas import tpu as pltpu
from jax.experimental.pallas import tpu_sc as plsc

NUM_ATOMS = 6_400_000
NUM_MOL = 100_000

LANES = 128
ROWS_PER_CHUNK = 16
TOTAL_ROWS = NUM_ATOMS // LANES
TOTAL_CHUNKS = TOTAL_ROWS // ROWS_PER_CHUNK
NC, NS = 2, 16
NW = NC * NS
BASE_CHUNKS = TOTAL_CHUNKS // NW
EXTRA = TOTAL_CHUNKS - BASE_CHUNKS * NW
ACC_PAD = 100_352
SLICE = ACC_PAD // NS


def _sc_body(vals_hbm, idx_hbm, out_hbm, vbuf, ibuf, zbuf, acc, sem):
    c = lax.axis_index("c")
    s = lax.axis_index("s")
    w = c * NS + s

    def _zero(i, carry):
        zbuf[pl.ds(i * 16, 16)] = jnp.zeros((16,), jnp.float32)
        return carry
    lax.fori_loop(0, SLICE // 16, _zero, None)
    pltpu.sync_copy(zbuf, acc.at[pl.ds(s * SLICE, SLICE)])
    plsc.subcore_barrier()

    def do_chunk(chunk_id):
        row0 = chunk_id * ROWS_PER_CHUNK
        pltpu.sync_copy(vals_hbm.at[pl.ds(row0, ROWS_PER_CHUNK)], vbuf)
        pltpu.sync_copy(idx_hbm.at[pl.ds(row0, ROWS_PER_CHUNK)], ibuf)
        copies = [
            pltpu.async_copy(vbuf.at[j], acc.at[ibuf.at[j]], sem, add=True)
            for j in range(ROWS_PER_CHUNK)
        ]
        for cp in copies:
            cp.wait()

    def _main(i, carry):
        do_chunk(w + i * NW)
        return carry
    lax.fori_loop(0, BASE_CHUNKS, _main, None)

    @pl.when(w < EXTRA)
    def _tail():
        do_chunk(w + BASE_CHUNKS * NW)

    plsc.subcore_barrier()
    pltpu.sync_copy(acc.at[pl.ds(s * SLICE, SLICE)],
                    out_hbm.at[c, pl.ds(s * SLICE, SLICE)])


_sc_call = functools.partial(
    pl.kernel,
    out_type=jax.ShapeDtypeStruct((NC, ACC_PAD), jnp.float32),
    mesh=plsc.VectorSubcoreMesh(core_axis_name="c", subcore_axis_name="s"),
    scratch_types=[
        pltpu.VMEM((ROWS_PER_CHUNK, LANES), jnp.float32),
        pltpu.VMEM((ROWS_PER_CHUNK, LANES), jnp.int32),
        pltpu.VMEM((SLICE,), jnp.float32),
        pltpu.VMEM_SHARED((ACC_PAD,), jnp.float32),
        pltpu.SemaphoreType.DMA,
    ],
)(_sc_body)


def kernel(atom_specific_values, index):
    vals2d = atom_specific_values.reshape(TOTAL_ROWS, LANES)
    idx2d = index.astype(jnp.int32).reshape(TOTAL_ROWS, LANES)
    partials = _sc_call(vals2d, idx2d)
    return (partials[0] + partials[1])[:NUM_MOL]


# speedup vs baseline: 35.2895x; 1.5383x over previous
import functools

import jax
import jax.numpy as jnp
from jax import lax
from jax.experimental import pallas as pl
from jax.experimental.pallas import tpu as pltpu
from jax.experimental.pallas import tpu_sc as plsc

NUM_ATOMS = 6_400_000
NUM_MOL = 100_000

CHUNK = 2048
TOTAL_CHUNKS = NUM_ATOMS // CHUNK
NC, NS = 2, 16
NW = NC * NS
BASE_CHUNKS = TOTAL_CHUNKS // NW
EXTRA = TOTAL_CHUNKS - BASE_CHUNKS * NW
ACC_PAD = 100_352
SLICE = ACC_PAD // NS


def _sc_body(vals_hbm, idx_hbm, out_hbm, vbuf0, vbuf1, ibuf0, ibuf1,
             zbuf, acc, sem0, sem1, sem_s):
    c = lax.axis_index("c")
    s = lax.axis_index("s")
    w = c * NS + s
    n = BASE_CHUNKS + jnp.where(w < EXTRA, 1, 0)

    def _zero(i, carry):
        zbuf[pl.ds(i * 16, 16)] = jnp.zeros((16,), jnp.float32)
        return carry
    lax.fori_loop(0, SLICE // 16, _zero, None)
    pltpu.sync_copy(zbuf, acc.at[pl.ds(s * SLICE, SLICE)])
    plsc.subcore_barrier()

    sems = (sem0, sem1)
    vbufs = (vbuf0, vbuf1)
    ibufs = (ibuf0, ibuf1)

    def off(i):
        return (w + i * NW) * CHUNK

    def start_load(i, b):
        pltpu.async_copy(vals_hbm.at[pl.ds(off(i), CHUNK)], vbufs[b],
                         sems[b])
        pltpu.async_copy(idx_hbm.at[pl.ds(off(i), CHUNK)], ibufs[b],
                         sems[b])

    def finish_chunk(i, b):
        pltpu.make_async_copy(vals_hbm.at[pl.ds(off(i), CHUNK)], vbufs[b],
                              sems[b]).wait()
        pltpu.make_async_copy(idx_hbm.at[pl.ds(off(i), CHUNK)], ibufs[b],
                              sems[b]).wait()
        pltpu.async_copy(vbufs[b], acc.at[ibufs[b]], sem_s,
                         add=True).wait()

    start_load(0, 0)

    def body(j, carry):
        i0 = 2 * j
        i1 = i0 + 1

        @pl.when(i1 < n)
        def _():
            start_load(i1, 1)

        finish_chunk(i0, 0)

        @pl.when(i1 < n)
        def _():
            @pl.when(i1 + 1 < n)
            def _():
                start_load(i1 + 1, 0)
            finish_chunk(i1, 1)

        return carry
    lax.fori_loop(0, (BASE_CHUNKS + 2) // 2, body, None)

    plsc.subcore_barrier()
    pltpu.sync_copy(acc.at[pl.ds(s * SLICE, SLICE)],
                    out_hbm.at[c, pl.ds(s * SLICE, SLICE)])


_sc_call = functools.partial(
    pl.kernel,
    out_type=jax.ShapeDtypeStruct((NC, ACC_PAD), jnp.float32),
    mesh=plsc.VectorSubcoreMesh(core_axis_name="c", subcore_axis_name="s"),
    scratch_types=[
        pltpu.VMEM((CHUNK,), jnp.float32),
        pltpu.VMEM((CHUNK,), jnp.float32),
        pltpu.VMEM((CHUNK,), jnp.int32),
        pltpu.VMEM((CHUNK,), jnp.int32),
        pltpu.VMEM((SLICE,), jnp.float32),
        pltpu.VMEM_SHARED((ACC_PAD,), jnp.float32),
        pltpu.SemaphoreType.DMA,
        pltpu.SemaphoreType.DMA,
        pltpu.SemaphoreType.DMA,
    ],
)(_sc_body)


def kernel(atom_specific_values, index):
    partials = _sc_call(atom_specific_values, index.astype(jnp.int32))
    return (partials[0] + partials[1])[:NUM_MOL]


# speedup vs baseline: 42.9021x; 1.2157x over previous
import functools

import jax
import jax.numpy as jnp
from jax import lax
from jax.experimental import pallas as pl
from jax.experimental.pallas import tpu as pltpu
from jax.experimental.pallas import tpu_sc as plsc

NUM_ATOMS = 6_400_000
NUM_MOL = 100_000

NC, NS = 2, 16
NW = NC * NS
APT = NUM_ATOMS // NW
CHUNK = 2048
FULL_CHUNKS = APT // CHUNK
TAIL = APT - FULL_CHUNKS * CHUNK
ACC_PAD = 100_352
SLICE = ACC_PAD // NS
PIECE = 128


def _sc_body(vals_hbm, idx_hbm, out_hbm, vbuf0, vbuf1, ibuf0, ibuf1,
             zbuf, ebuf, pbuf, idbuf, wacc, acc, sem0, sem1, sem_s):
    c = lax.axis_index("c")
    s = lax.axis_index("s")
    w = c * NS + s
    a0 = w * APT

    lane = lax.iota(jnp.int32, 16)
    lt15 = lane < 15
    is15 = lane == 15
    zeros16 = jnp.zeros((16,), jnp.float32)

    def _zero(i, carry):
        zbuf[pl.ds(i * 16, 16)] = zeros16
        return carry
    lax.fori_loop(0, SLICE // 16, _zero, None)
    pltpu.sync_copy(zbuf, acc.at[pl.ds(s * SLICE, SLICE)])

    pltpu.sync_copy(idx_hbm.at[pl.ds(a0, 8)], ebuf.at[pl.ds(0, 8)])
    pltpu.sync_copy(idx_hbm.at[pl.ds(a0 + APT - 8, 8)], ebuf.at[pl.ds(8, 8)])
    ev = ebuf[pl.ds(0, 16)]
    mlo = ev[0]
    mhi = ev[15]
    npieces = (mhi - mlo) // PIECE + 1

    def _wzero(k, carry):
        base = mlo + k * PIECE
        for j in range(PIECE // 16):
            wacc[pl.ds(base + 16 * j, 16)] = zeros16
        return carry
    lax.fori_loop(0, npieces, _wzero, None)

    plsc.subcore_barrier()

    sems = (sem0, sem1)
    vbufs = (vbuf0, vbuf1)
    ibufs = (ibuf0, ibuf1)

    def start_load(i, b, count):
        off = a0 + i * CHUNK
        pltpu.async_copy(vals_hbm.at[pl.ds(off, count)],
                         vbufs[b].at[pl.ds(0, count)], sems[b])
        pltpu.async_copy(idx_hbm.at[pl.ds(off, count)],
                         ibufs[b].at[pl.ds(0, count)], sems[b])

    def wait_load(i, b, count):
        off = a0 + i * CHUNK
        pltpu.make_async_copy(vals_hbm.at[pl.ds(off, count)],
                              vbufs[b].at[pl.ds(0, count)], sems[b]).wait()
        pltpu.make_async_copy(idx_hbm.at[pl.ds(off, count)],
                              ibufs[b].at[pl.ds(0, count)], sems[b]).wait()

    def vreg_step(vb, ib, m):
        v = vb[pl.ds(16 * m, 16)]
        ic = ib[pl.ds(16 * m, 16)]
        inx = ib[pl.ds(16 * m + 1, 16)]
        p = plsc.cumsum(v)
        trans = (ic != inx) & lt15
        endm = trans | is15
        plsc.addupdate_scatter(wacc, [ic], p, mask=endm)
        plsc.addupdate_scatter(wacc, [inx], -p, mask=trans)

    def process(b, nvregs):
        vb, ib = vbufs[b], ibufs[b]

        def _inner(q, carry):
            for r in range(4):
                vreg_step(vb, ib, 4 * q + r)
            return carry
        lax.fori_loop(0, nvregs // 4, _inner, None)

    start_load(0, 0, CHUNK)

    def _pair(j, carry):
        i0 = 2 * j
        start_load(i0 + 1, 1, CHUNK)
        wait_load(i0, 0, CHUNK)
        process(0, CHUNK // 16)
        start_load(i0 + 2, 0, CHUNK)
        wait_load(i0 + 1, 1, CHUNK)
        process(1, CHUNK // 16)
        return carry
    lax.fori_loop(0, FULL_CHUNKS // 2, _pair, None)

    start_load(FULL_CHUNKS, 1, TAIL)
    wait_load(FULL_CHUNKS - 1, 0, CHUNK)
    process(0, CHUNK // 16)
    wait_load(FULL_CHUNKS, 1, TAIL)
    process(1, TAIL // 16)

    def _merge(k, carry):
        base = mlo + k * PIECE
        for j in range(PIECE // 16):
            pbuf[pl.ds(16 * j, 16)] = wacc[pl.ds(base + 16 * j, 16)]
            idbuf[pl.ds(16 * j, 16)] = base + 16 * j + lane
        pltpu.async_copy(pbuf, acc.at[idbuf], sem_s, add=True).wait()
        return carry
    lax.fori_loop(0, npieces, _merge, None)

    plsc.subcore_barrier()
    pltpu.sync_copy(acc.at[pl.ds(s * SLICE, SLICE)],
                    out_hbm.at[c, pl.ds(s * SLICE, SLICE)])


_sc_call = functools.partial(
    pl.kernel,
    out_type=jax.ShapeDtypeStruct((NC, ACC_PAD), jnp.float32),
    mesh=plsc.VectorSubcoreMesh(core_axis_name="c", subcore_axis_name="s"),
    compiler_params=pltpu.CompilerParams(needs_layout_passes=False),
    scratch_types=[
        pltpu.VMEM((CHUNK,), jnp.float32),
        pltpu.VMEM((CHUNK,), jnp.float32),
        pltpu.VMEM((CHUNK + 16,), jnp.int32),
        pltpu.VMEM((CHUNK + 16,), jnp.int32),
        pltpu.VMEM((SLICE,), jnp.float32),
        pltpu.VMEM((16,), jnp.int32),
        pltpu.VMEM((PIECE,), jnp.float32),
        pltpu.VMEM((PIECE,), jnp.int32),
        pltpu.VMEM((ACC_PAD,), jnp.float32),
        pltpu.VMEM_SHARED((ACC_PAD,), jnp.float32),
        pltpu.SemaphoreType.DMA,
        pltpu.SemaphoreType.DMA,
        pltpu.SemaphoreType.DMA,
    ],
)(_sc_body)


def kernel(atom_specific_values, index):
    partials = _sc_call(atom_specific_values, index.astype(jnp.int32))
    return (partials[0] + partials[1])[:NUM_MOL]
